# Initial kernel scaffold; baseline (speedup 1.0000x reference)
import jax
import jax.numpy as jnp
from jax.experimental import pallas as pl


def kernel(edge_attr, W0, W1, W2):
    raise NotImplementedError("write your pallas kernel here")


# speedup vs baseline: 1.0815x; 1.0815x over previous
import functools

import jax
import jax.numpy as jnp
from jax import lax
from jax.experimental import pallas as pl
from jax.experimental.pallas import tpu as pltpu
from jax.experimental.pallas import tpu_sc as plsc

_NC = 2
_NS = 16
_NW = _NC * _NS
_CH = 80


def _make_sc_lookup(E, D):
    per_w = E // _NW
    n_chunks = per_w // _CH
    mesh = plsc.VectorSubcoreMesh(core_axis_name="c", subcore_axis_name="s")

    @functools.partial(
        pl.kernel,
        mesh=mesh,
        out_type=jax.ShapeDtypeStruct((E, D), jnp.float32),
        scratch_types=[
            pltpu.VMEM((_CH,), jnp.int32),
            pltpu.VMEM((_CH,), jnp.int32),
            pltpu.VMEM((_CH,), jnp.int32),
            pltpu.VMEM((_CH,), jnp.int32),
            pltpu.VMEM((_CH, D), jnp.float32),
            pltpu.SemaphoreType.DMA,
        ],
    )
    def lookup(lut_hbm, a0_hbm, a1_hbm, a2_hbm, out_hbm,
               a0_v, a1_v, a2_v, idx_v, rows_v, sem):
        wid = lax.axis_index("s") * _NC + lax.axis_index("c")
        tile_base = wid * per_w

        def body(i, carry):
            base = tile_base + i * _CH
            pltpu.sync_copy(a0_hbm.at[pl.ds(base, _CH)], a0_v)
            pltpu.sync_copy(a1_hbm.at[pl.ds(base, _CH)], a1_v)
            pltpu.sync_copy(a2_hbm.at[pl.ds(base, _CH)], a2_v)
            for j in range(_CH // 16):
                s = pl.ds(j * 16, 16)
                idx_v[s] = a0_v[s] * 12 + a1_v[s] * 2 + a2_v[s]
            pltpu.async_copy(lut_hbm.at[idx_v], rows_v, sem).wait()
            pltpu.sync_copy(rows_v, out_hbm.at[pl.ds(base, _CH)])
            return carry

        lax.fori_loop(0, n_chunks, body, 0)

    return lookup


def kernel(edge_attr, W0, W1, W2):
    E = edge_attr.shape[0]
    D = W0.shape[1]
    lut = (W0[:, None, None, :] + W1[None, :, None, :]
           + W2[None, None, :, :]).reshape(-1, D)
    lut = jnp.pad(lut, ((0, 4), (0, 0)))
    ea = edge_attr.astype(jnp.int32)
    a0 = ea[:, 0]
    a1 = ea[:, 1]
    a2 = ea[:, 2]
    return _make_sc_lookup(E, D)(lut, a0, a1, a2)


# speedup vs baseline: 1.0900x; 1.0079x over previous
import functools

import jax
import jax.numpy as jnp
from jax import lax
from jax.experimental import pallas as pl
from jax.experimental.pallas import tpu as pltpu
from jax.experimental.pallas import tpu_sc as plsc

_NC = 2
_NS = 16
_NW = _NC * _NS
_CH = 80


def _make_sc_lookup(E, D):
    per_w = E // _NW
    n_chunks = per_w // _CH
    n_outer = n_chunks // 4
    tail = n_chunks - n_outer * 4
    mesh = plsc.VectorSubcoreMesh(core_axis_name="c", subcore_axis_name="s")

    @functools.partial(
        pl.kernel,
        mesh=mesh,
        out_type=jax.ShapeDtypeStruct((E, D), jnp.float32),
        scratch_types=[
            pltpu.VMEM((per_w,), jnp.int32),
            pltpu.VMEM((per_w,), jnp.int32),
            pltpu.VMEM((per_w,), jnp.int32),
            pltpu.VMEM((per_w,), jnp.int32),
            pltpu.VMEM((_CH, D), jnp.float32),
            pltpu.VMEM((_CH, D), jnp.float32),
            pltpu.VMEM((_CH, D), jnp.float32),
            pltpu.VMEM((_CH, D), jnp.float32),
            pltpu.SemaphoreType.DMA,
            pltpu.SemaphoreType.DMA,
        ],
    )
    def lookup(lut_hbm, a0_hbm, a1_hbm, a2_hbm, out_hbm,
               a0_v, a1_v, a2_v, idx_v, r0, r1, r2, r3, gsem, wsem):
        rows = (r0, r1, r2, r3)
        wid = lax.axis_index("s") * _NC + lax.axis_index("c")
        tile_base = wid * per_w

        pltpu.sync_copy(a0_hbm.at[pl.ds(tile_base, per_w)], a0_v)
        pltpu.sync_copy(a1_hbm.at[pl.ds(tile_base, per_w)], a1_v)
        pltpu.sync_copy(a2_hbm.at[pl.ds(tile_base, per_w)], a2_v)

        def idx_body(c, carry):
            for j in range(_CH // 16):
                s = pl.ds(c * _CH + j * 16, 16)
                idx_v[s] = a0_v[s] * 12 + a1_v[s] * 2 + a2_v[s]
            return carry

        lax.fori_loop(0, n_chunks, idx_body, 0)

        def gather(c, buf):
            return pltpu.async_copy(
                lut_hbm.at[idx_v.at[pl.ds(c * _CH, _CH)]], buf, gsem)

        def wait_gather(c, buf):
            pltpu.make_async_copy(
                lut_hbm.at[idx_v.at[pl.ds(c * _CH, _CH)]], buf, gsem).wait()

        def writeback(c, buf):
            return pltpu.async_copy(
                buf, out_hbm.at[pl.ds(tile_base + c * _CH, _CH)], wsem)

        def wait_writeback(c, buf):
            pltpu.make_async_copy(
                buf, out_hbm.at[pl.ds(tile_base + c * _CH, _CH)], wsem).wait()

        def outer_body(t, carry):
            c0 = t * 4
            gather(c0 + 0, r0)
            gather(c0 + 1, r1)

            @pl.when(t > 0)
            def _drain_prev():
                wait_writeback(c0 - 2, r2)
                wait_writeback(c0 - 1, r3)

            wait_gather(c0 + 0, r0)
            wait_gather(c0 + 1, r1)
            writeback(c0 + 0, r0)
            writeback(c0 + 1, r1)

            gather(c0 + 2, r2)
            gather(c0 + 3, r3)
            wait_writeback(c0 + 0, r0)
            wait_writeback(c0 + 1, r1)
            wait_gather(c0 + 2, r2)
            wait_gather(c0 + 3, r3)
            writeback(c0 + 2, r2)
            writeback(c0 + 3, r3)
            return carry

        lax.fori_loop(0, n_outer, outer_body, 0)

        wait_writeback(n_outer * 4 - 2, r2)
        wait_writeback(n_outer * 4 - 1, r3)

        for t in range(tail):
            c = n_outer * 4 + t
            gather(c, r0).wait()
            pltpu.sync_copy(r0, out_hbm.at[pl.ds(tile_base + c * _CH, _CH)])

    return lookup


def kernel(edge_attr, W0, W1, W2):
    E = edge_attr.shape[0]
    D = W0.shape[1]
    lut = (W0[:, None, None, :] + W1[None, :, None, :]
           + W2[None, None, :, :]).reshape(-1, D)
    lut = jnp.pad(lut, ((0, 4), (0, 0)))
    ea = edge_attr.astype(jnp.int32)
    return _make_sc_lookup(E, D)(lut, ea[:, 0], ea[:, 1], ea[:, 2])


# speedup vs baseline: 18.6494x; 17.1092x over previous
import functools

import jax
import jax.numpy as jnp
from jax import lax
from jax.experimental import pallas as pl
from jax.experimental.pallas import tpu as pltpu
from jax.experimental.pallas import tpu_sc as plsc

_NC = 2
_NS = 16
_NW = _NC * _NS
_CH = 80


def _make_sc_lookup(E, D):
    per_w = E // _NW
    n_chunks = per_w // _CH
    n_outer = n_chunks // 4
    tail = n_chunks - n_outer * 4
    mesh = plsc.VectorSubcoreMesh(core_axis_name="c", subcore_axis_name="s")

    @functools.partial(
        pl.kernel,
        mesh=mesh,
        out_type=jax.ShapeDtypeStruct((E, D), jnp.float32),
        scratch_types=[
            pltpu.VMEM((per_w,), jnp.int32),
            pltpu.VMEM((per_w,), jnp.int32),
            pltpu.VMEM((per_w,), jnp.int32),
            pltpu.VMEM((per_w,), jnp.int32),
            pltpu.VMEM((_CH, D), jnp.float32),
            pltpu.VMEM((_CH, D), jnp.float32),
            pltpu.VMEM((_CH, D), jnp.float32),
            pltpu.VMEM((_CH, D), jnp.float32),
            pltpu.VMEM_SHARED((64, D), jnp.float32),
            pltpu.SemaphoreType.DMA,
            pltpu.SemaphoreType.DMA,
        ],
    )
    def lookup(lut_hbm, a0_hbm, a1_hbm, a2_hbm, out_hbm,
               a0_v, a1_v, a2_v, idx_v, r0, r1, r2, r3, lut_sp, gsem, wsem):
        sid = lax.axis_index("s")
        wid = sid * _NC + lax.axis_index("c")
        tile_base = wid * per_w

        @pl.when(sid == 0)
        def _stage_lut():
            pltpu.sync_copy(lut_hbm, lut_sp)

        pltpu.sync_copy(a0_hbm.at[pl.ds(tile_base, per_w)], a0_v)
        pltpu.sync_copy(a1_hbm.at[pl.ds(tile_base, per_w)], a1_v)
        pltpu.sync_copy(a2_hbm.at[pl.ds(tile_base, per_w)], a2_v)

        def idx_body(c, carry):
            for j in range(_CH // 16):
                s = pl.ds(c * _CH + j * 16, 16)
                idx_v[s] = a0_v[s] * 12 + a1_v[s] * 2 + a2_v[s]
            return carry

        lax.fori_loop(0, n_chunks, idx_body, 0)

        plsc.subcore_barrier()

        def gather(c, buf):
            return pltpu.async_copy(
                lut_sp.at[idx_v.at[pl.ds(c * _CH, _CH)]], buf, gsem)

        def wait_gather(c, buf):
            pltpu.make_async_copy(
                lut_sp.at[idx_v.at[pl.ds(c * _CH, _CH)]], buf, gsem).wait()

        def writeback(c, buf):
            return pltpu.async_copy(
                buf, out_hbm.at[pl.ds(tile_base + c * _CH, _CH)], wsem)

        def wait_writeback(c, buf):
            pltpu.make_async_copy(
                buf, out_hbm.at[pl.ds(tile_base + c * _CH, _CH)], wsem).wait()

        def outer_body(t, carry):
            c0 = t * 4
            gather(c0 + 0, r0)
            gather(c0 + 1, r1)

            @pl.when(t > 0)
            def _drain_prev():
                wait_writeback(c0 - 2, r2)
                wait_writeback(c0 - 1, r3)

            wait_gather(c0 + 0, r0)
            wait_gather(c0 + 1, r1)
            writeback(c0 + 0, r0)
            writeback(c0 + 1, r1)

            gather(c0 + 2, r2)
            gather(c0 + 3, r3)
            wait_writeback(c0 + 0, r0)
            wait_writeback(c0 + 1, r1)
            wait_gather(c0 + 2, r2)
            wait_gather(c0 + 3, r3)
            writeback(c0 + 2, r2)
            writeback(c0 + 3, r3)
            return carry

        lax.fori_loop(0, n_outer, outer_body, 0)

        wait_writeback(n_outer * 4 - 2, r2)
        wait_writeback(n_outer * 4 - 1, r3)

        for t in range(tail):
            c = n_outer * 4 + t
            gather(c, r0).wait()
            pltpu.sync_copy(r0, out_hbm.at[pl.ds(tile_base + c * _CH, _CH)])

    return lookup


def kernel(edge_attr, W0, W1, W2):
    E = edge_attr.shape[0]
    D = W0.shape[1]
    lut = (W0[:, None, None, :] + W1[None, :, None, :]
           + W2[None, None, :, :]).reshape(-1, D)
    lut = jnp.pad(lut, ((0, 4), (0, 0)))
    ea = edge_attr.astype(jnp.int32)
    return _make_sc_lookup(E, D)(lut, ea[:, 0], ea[:, 1], ea[:, 2])


# speedup vs baseline: 18.8502x; 1.0108x over previous
import functools

import jax
import jax.numpy as jnp
from jax import lax
from jax.experimental import pallas as pl
from jax.experimental.pallas import tpu as pltpu
from jax.experimental.pallas import tpu_sc as plsc

_NC = 2
_NS = 16
_NW = _NC * _NS
_CH = 80


def _make_sc_lookup(E, D):
    per_w = E // _NW
    n_chunks = per_w // _CH
    n_outer = n_chunks // 4
    tail = n_chunks - n_outer * 4
    mesh = plsc.VectorSubcoreMesh(core_axis_name="c", subcore_axis_name="s")

    @functools.partial(
        pl.kernel,
        mesh=mesh,
        out_type=jax.ShapeDtypeStruct((E, D), jnp.float32),
        scratch_types=[
            pltpu.VMEM((per_w,), jnp.int32),
            pltpu.VMEM((per_w,), jnp.int32),
            pltpu.VMEM((per_w,), jnp.int32),
            pltpu.VMEM((per_w,), jnp.int32),
            pltpu.VMEM((_CH, D), jnp.float32),
            pltpu.VMEM((_CH, D), jnp.float32),
            pltpu.VMEM((_CH, D), jnp.float32),
            pltpu.VMEM((_CH, D), jnp.float32),
            pltpu.VMEM_SHARED((64, D), jnp.float32),
            pltpu.SemaphoreType.DMA,
            pltpu.SemaphoreType.DMA,
        ],
    )
    def lookup(lut_hbm, a0_hbm, a1_hbm, a2_hbm, out_hbm,
               a0_v, a1_v, a2_v, idx_v, r0, r1, r2, r3, lut_sp, gsem, wsem):
        sid = lax.axis_index("s")
        wid = sid * _NC + lax.axis_index("c")
        tile_base = wid * per_w

        @pl.when(sid == 0)
        def _stage_lut():
            pltpu.sync_copy(lut_hbm, lut_sp)

        pltpu.sync_copy(a0_hbm.at[pl.ds(tile_base, per_w)], a0_v)
        pltpu.sync_copy(a1_hbm.at[pl.ds(tile_base, per_w)], a1_v)
        pltpu.sync_copy(a2_hbm.at[pl.ds(tile_base, per_w)], a2_v)

        def idx_group(g):
            def body(j, carry):
                s = pl.ds(g * (4 * _CH) + j * 16, 16)
                idx_v[s] = a0_v[s] * 12 + a1_v[s] * 2 + a2_v[s]
                return carry
            lax.fori_loop(0, 4 * _CH // 16, body, 0)

        idx_group(0)
        for t in range(tail):
            c = n_outer * 4 + t
            def tail_body(j, carry, c=c):
                s = pl.ds(c * _CH + j * 16, 16)
                idx_v[s] = a0_v[s] * 12 + a1_v[s] * 2 + a2_v[s]
                return carry
            lax.fori_loop(0, _CH // 16, tail_body, 0)

        plsc.subcore_barrier()

        def gather(c, buf):
            return pltpu.async_copy(
                lut_sp.at[idx_v.at[pl.ds(c * _CH, _CH)]], buf, gsem)

        def wait_gather(c, buf):
            pltpu.make_async_copy(
                lut_sp.at[idx_v.at[pl.ds(c * _CH, _CH)]], buf, gsem).wait()

        def writeback(c, buf):
            return pltpu.async_copy(
                buf, out_hbm.at[pl.ds(tile_base + c * _CH, _CH)], wsem)

        def wait_writeback(c, buf):
            pltpu.make_async_copy(
                buf, out_hbm.at[pl.ds(tile_base + c * _CH, _CH)], wsem).wait()

        def outer_body(t, carry):
            c0 = t * 4
            gather(c0 + 0, r0)
            gather(c0 + 1, r1)

            @pl.when(t + 1 < n_outer)
            def _precompute_next_idx():
                idx_group(t + 1)

            @pl.when(t > 0)
            def _drain_prev():
                wait_writeback(c0 - 2, r2)
                wait_writeback(c0 - 1, r3)

            wait_gather(c0 + 0, r0)
            wait_gather(c0 + 1, r1)
            writeback(c0 + 0, r0)
            writeback(c0 + 1, r1)

            gather(c0 + 2, r2)
            gather(c0 + 3, r3)
            wait_writeback(c0 + 0, r0)
            wait_writeback(c0 + 1, r1)
            wait_gather(c0 + 2, r2)
            wait_gather(c0 + 3, r3)
            writeback(c0 + 2, r2)
            writeback(c0 + 3, r3)
            return carry

        lax.fori_loop(0, n_outer, outer_body, 0)

        wait_writeback(n_outer * 4 - 2, r2)
        wait_writeback(n_outer * 4 - 1, r3)

        for t in range(tail):
            c = n_outer * 4 + t
            gather(c, r0).wait()
            pltpu.sync_copy(r0, out_hbm.at[pl.ds(tile_base + c * _CH, _CH)])

    return lookup


def kernel(edge_attr, W0, W1, W2):
    E = edge_attr.shape[0]
    D = W0.shape[1]
    lut = (W0[:, None, None, :] + W1[None, :, None, :]
           + W2[None, None, :, :]).reshape(-1, D)
    lut = jnp.pad(lut, ((0, 4), (0, 0)))
    ea = edge_attr.astype(jnp.int32)
    return _make_sc_lookup(E, D)(lut, ea[:, 0], ea[:, 1], ea[:, 2])
